# baseline (device time: 41010 ns/iter reference)
import jax
import jax.numpy as jnp
from jax import lax
from jax.experimental import pallas as pl
from jax.experimental.pallas import tpu as pltpu

N_DEV = 4
B_LOC = 2
SQ = 256
SKV = 256
H_GRP = 4
DH = 64
D_MODEL = 512
GRP_W = H_GRP * DH
WINDOW = 128
SCALE = 0.125


def kernel(x, Wq, K_ext, V_ext, Wo):
    K2 = K_ext.reshape(8, SKV, 16 * DH)
    V2 = V_ext.reshape(8, SKV, 16 * DH)

    def body(x_ref, wq_ref, k_ref, v_ref, wo_ref, out_ref,
             comm_wq, comm_wo, send_wq, recv_wq, send_wo, recv_wo):
        my_pos = lax.axis_index("i")
        right = lax.rem(my_pos + 1, N_DEV)
        left = lax.rem(my_pos + N_DEV - 1, N_DEV)

        barrier_sem = pltpu.get_barrier_semaphore()
        for nbr in (left, right):
            pl.semaphore_signal(
                barrier_sem, inc=1,
                device_id=(nbr,), device_id_type=pl.DeviceIdType.MESH,
            )
        pl.semaphore_wait(barrier_sem, 2)

        comm_wq[0] = wq_ref[...].astype(jnp.bfloat16)
        comm_wo[0] = wo_ref[...].astype(jnp.bfloat16)

        x_bf = [x_ref[b].astype(jnp.bfloat16) for b in range(B_LOC)]

        qi = lax.broadcasted_iota(jnp.int32, (SQ, SKV), 0)
        ki = lax.broadcasted_iota(jnp.int32, (SQ, SKV), 1)
        mask = jnp.abs(qi - ki) <= WINDOW

        for h in range(N_DEV):
            origin = lax.rem(my_pos + (N_DEV - h), N_DEV)

            if h < N_DEV - 1:
                rdma_wq = pltpu.make_async_remote_copy(
                    src_ref=comm_wq.at[h], dst_ref=comm_wq.at[h + 1],
                    send_sem=send_wq.at[h], recv_sem=recv_wq.at[h],
                    device_id=(right,), device_id_type=pl.DeviceIdType.MESH,
                )
                rdma_wo = pltpu.make_async_remote_copy(
                    src_ref=comm_wo.at[h], dst_ref=comm_wo.at[h + 1],
                    send_sem=send_wo.at[h], recv_sem=recv_wo.at[h],
                    device_id=(right,), device_id_type=pl.DeviceIdType.MESH,
                )
                rdma_wq.start()
                rdma_wo.start()

            wq_chunk = comm_wq[h]
            wo_chunk = comm_wo[h]
            col0 = origin * GRP_W
            for b in range(B_LOC):
                gb = my_pos * B_LOC + b
                q_grp = lax.dot_general(
                    x_bf[b], wq_chunk, (((1,), (0,)), ((), ())),
                    preferred_element_type=jnp.float32,
                ).astype(jnp.bfloat16)
                k_grp = k_ref[gb, :, pl.ds(col0, GRP_W)].astype(jnp.bfloat16)
                v_grp = v_ref[gb, :, pl.ds(col0, GRP_W)].astype(jnp.bfloat16)

                ctx_list = []
                for hh in range(H_GRP):
                    sl = slice(hh * DH, (hh + 1) * DH)
                    s = lax.dot_general(
                        q_grp[:, sl], k_grp[:, sl], (((1,), (1,)), ((), ())),
                        preferred_element_type=jnp.float32,
                    ) * SCALE
                    s = jnp.where(mask, s, -1e9)
                    m = jnp.max(s, axis=1, keepdims=True)
                    e = jnp.exp(s - m)
                    w = (e / jnp.sum(e, axis=1, keepdims=True)).astype(
                        jnp.bfloat16)
                    ctx_list.append(
                        lax.dot_general(
                            w, v_grp[:, sl], (((1,), (0,)), ((), ())),
                            preferred_element_type=jnp.float32,
                        ).astype(jnp.bfloat16))
                ctx_grp = jnp.concatenate(ctx_list, axis=1)
                contrib = lax.dot_general(
                    ctx_grp, wo_chunk, (((1,), (0,)), ((), ())),
                    preferred_element_type=jnp.float32,
                )
                if h == 0:
                    out_ref[b] = contrib
                else:
                    out_ref[b] = out_ref[b] + contrib

            if h < N_DEV - 1:
                rdma_wq.wait()
                rdma_wo.wait()

    return pl.pallas_call(
        body,
        out_shape=jax.ShapeDtypeStruct((B_LOC, SQ, D_MODEL), jnp.float32),
        in_specs=[pl.BlockSpec(memory_space=pltpu.VMEM)] * 5,
        out_specs=pl.BlockSpec(memory_space=pltpu.VMEM),
        scratch_shapes=[
            pltpu.VMEM((N_DEV, D_MODEL, GRP_W), jnp.bfloat16),
            pltpu.VMEM((N_DEV, GRP_W, D_MODEL), jnp.bfloat16),
            pltpu.SemaphoreType.DMA((N_DEV - 1,)),
            pltpu.SemaphoreType.DMA((N_DEV - 1,)),
            pltpu.SemaphoreType.DMA((N_DEV - 1,)),
            pltpu.SemaphoreType.DMA((N_DEV - 1,)),
        ],
        compiler_params=pltpu.CompilerParams(collective_id=0),
    )(x, Wq, K2, V2, Wo)


# device time: 36815 ns/iter; 1.1139x vs baseline; 1.1139x over previous
import jax
import jax.numpy as jnp
from jax import lax
from jax.experimental import pallas as pl
from jax.experimental.pallas import tpu as pltpu

N_DEV = 4
B_LOC = 2
SQ = 256
SKV = 256
H_GRP = 4
DH = 64
D_MODEL = 512
GRP_W = H_GRP * DH
WINDOW = 128
SCALE = 0.125


def kernel(x, Wq, K_ext, V_ext, Wo):
    K2 = K_ext.reshape(8, SKV, 16 * DH)
    V2 = V_ext.reshape(8, SKV, 16 * DH)

    def body(x_ref, wq_ref, k_ref, v_ref, wo_ref, out_ref,
             comm, send_sems, recv_sems):
        my_pos = lax.axis_index("i")
        right = lax.rem(my_pos + 1, N_DEV)
        left = lax.rem(my_pos + N_DEV - 1, N_DEV)
        across = lax.rem(my_pos + 2, N_DEV)

        barrier_sem = pltpu.get_barrier_semaphore()
        for nbr in (left, right, across):
            pl.semaphore_signal(
                barrier_sem, inc=1,
                device_id=(nbr,), device_id_type=pl.DeviceIdType.MESH,
            )
        pl.semaphore_wait(barrier_sem, 3)

        comm[0, 0] = wq_ref[...].astype(jnp.bfloat16)
        comm[0, 1] = wo_ref[...].astype(jnp.bfloat16)

        def mk(i, dst_slot, dev):
            return pltpu.make_async_remote_copy(
                src_ref=comm.at[0], dst_ref=comm.at[dst_slot],
                send_sem=send_sems.at[i], recv_sem=recv_sems.at[i],
                device_id=(dev,), device_id_type=pl.DeviceIdType.MESH,
            )

        sends = [mk(2, 3, across), mk(0, 1, right), mk(1, 2, left)]
        for d in sends:
            d.start()

        x_bf = (x_ref[...] * SCALE).astype(jnp.bfloat16)
        x2 = jnp.concatenate([x_bf[0], x_bf[1]], axis=0)

        qi = lax.broadcasted_iota(jnp.int32, (SQ, SKV), 0)
        ki = lax.broadcasted_iota(jnp.int32, (SQ, SKV), 1)
        mask = jnp.abs(qi - ki) <= WINDOW

        def group_contrib(origin, slot):
            col0 = origin * GRP_W
            wqt_chunk = comm[slot, 0]
            wo_chunk = comm[slot, 1]
            q2 = lax.dot_general(
                x2, wqt_chunk, (((1,), (1,)), ((), ())),
                preferred_element_type=jnp.float32,
            ).astype(jnp.bfloat16)
            ctx = []
            for b in range(B_LOC):
                gb = my_pos * B_LOC + b
                k_grp = k_ref[gb, :, pl.ds(col0, GRP_W)].astype(jnp.bfloat16)
                v_grp = v_ref[gb, :, pl.ds(col0, GRP_W)].astype(jnp.bfloat16)
                q_grp = q2[b * SQ:(b + 1) * SQ]
                for hh in range(H_GRP):
                    sl = slice(hh * DH, (hh + 1) * DH)
                    s = lax.dot_general(
                        q_grp[:, sl], k_grp[:, sl], (((1,), (1,)), ((), ())),
                        preferred_element_type=jnp.float32,
                    )
                    e = jnp.where(mask, jnp.exp(s), 0.0)
                    w = (e * (1.0 / jnp.sum(e, axis=1, keepdims=True))
                         ).astype(jnp.bfloat16)
                    ctx.append(lax.dot_general(
                        w, v_grp[:, sl], (((1,), (0,)), ((), ())),
                        preferred_element_type=jnp.float32,
                    ).astype(jnp.bfloat16))
            ctx2 = jnp.concatenate(
                [jnp.concatenate(ctx[:H_GRP], axis=1),
                 jnp.concatenate(ctx[H_GRP:], axis=1)], axis=0)
            return lax.dot_general(
                ctx2, wo_chunk, (((1,), (0,)), ((), ())),
                preferred_element_type=jnp.float32,
            )

        acc = group_contrib(my_pos, 0)

        sends[1].wait_recv()
        acc = acc + group_contrib(left, 1)

        sends[2].wait_recv()
        acc = acc + group_contrib(right, 2)

        sends[0].wait_recv()
        acc = acc + group_contrib(across, 3)

        out_ref[0] = acc[:SQ]
        out_ref[1] = acc[SQ:]

        for d in sends:
            d.wait_send()

    return pl.pallas_call(
        body,
        out_shape=jax.ShapeDtypeStruct((B_LOC, SQ, D_MODEL), jnp.float32),
        in_specs=[pl.BlockSpec(memory_space=pltpu.VMEM)] * 5,
        out_specs=pl.BlockSpec(memory_space=pltpu.VMEM),
        scratch_shapes=[
            pltpu.VMEM((N_DEV, 2, GRP_W, D_MODEL), jnp.bfloat16),
            pltpu.SemaphoreType.DMA((3,)),
            pltpu.SemaphoreType.DMA((3,)),
        ],
        compiler_params=pltpu.CompilerParams(collective_id=0),
    )(x, Wq.T, K2, V2, Wo)


# device time: 31832 ns/iter; 1.2883x vs baseline; 1.1565x over previous
import jax
import jax.numpy as jnp
from jax import lax
from jax.experimental import pallas as pl
from jax.experimental.pallas import tpu as pltpu

N_DEV = 4
B_LOC = 2
SQ = 256
SKV = 256
H_GRP = 4
DH = 64
D_MODEL = 512
GRP_W = H_GRP * DH
WINDOW = 128
SCALE = 0.125


def kernel(x, Wq, K_ext, V_ext, Wo):
    K2 = K_ext.reshape(8, SKV, 16 * DH)
    V2 = V_ext.reshape(8, SKV, 16 * DH)

    def body(x_ref, wq_ref, k_ref, v_ref, wo_ref, out_ref,
             comm_wq, comm_wo, send_wq, recv_wq, send_wo, recv_wo):
        my_pos = lax.axis_index("i")
        right = lax.rem(my_pos + 1, N_DEV)
        left = lax.rem(my_pos + N_DEV - 1, N_DEV)
        across = lax.rem(my_pos + 2, N_DEV)

        barrier_sem = pltpu.get_barrier_semaphore()
        for nbr in (left, right, across):
            pl.semaphore_signal(
                barrier_sem, inc=1,
                device_id=(nbr,), device_id_type=pl.DeviceIdType.MESH,
            )
        pl.semaphore_wait(barrier_sem, 3)

        comm_wq[0] = wq_ref[...].astype(jnp.bfloat16)
        comm_wo[0] = wo_ref[...].astype(jnp.bfloat16)

        def mk(comm, send_sems, recv_sems, i, dst_slot, dev):
            return pltpu.make_async_remote_copy(
                src_ref=comm.at[0], dst_ref=comm.at[dst_slot],
                send_sem=send_sems.at[i], recv_sem=recv_sems.at[i],
                device_id=(dev,), device_id_type=pl.DeviceIdType.MESH,
            )

        sends = [mk(comm_wq, send_wq, recv_wq, 0, 1, right),
                 mk(comm_wo, send_wo, recv_wo, 0, 1, right),
                 mk(comm_wq, send_wq, recv_wq, 1, 2, left),
                 mk(comm_wo, send_wo, recv_wo, 1, 2, left),
                 mk(comm_wq, send_wq, recv_wq, 2, 3, across),
                 mk(comm_wo, send_wo, recv_wo, 2, 3, across)]
        for d in sends:
            d.start()

        x_bf = (x_ref[...] * SCALE).astype(jnp.bfloat16)
        x2 = jnp.concatenate([x_bf[0], x_bf[1]], axis=0)

        qi = lax.broadcasted_iota(jnp.int32, (SQ, SKV), 0)
        ki = lax.broadcasted_iota(jnp.int32, (SQ, SKV), 1)
        mask = jnp.abs(qi - ki) <= WINDOW

        def group_contrib(origin, wq_chunk, wo_chunk):
            col0 = origin * GRP_W
            q2 = lax.dot_general(
                x2, wq_chunk, (((1,), (0,)), ((), ())),
                preferred_element_type=jnp.float32,
            ).astype(jnp.bfloat16)
            ctx = []
            for b in range(B_LOC):
                gb = my_pos * B_LOC + b
                k_grp = k_ref[gb, :, pl.ds(col0, GRP_W)].astype(jnp.bfloat16)
                v_grp = v_ref[gb, :, pl.ds(col0, GRP_W)].astype(jnp.bfloat16)
                q_grp = q2[b * SQ:(b + 1) * SQ]
                for hh in range(H_GRP):
                    sl = slice(hh * DH, (hh + 1) * DH)
                    s = lax.dot_general(
                        q_grp[:, sl], k_grp[:, sl], (((1,), (1,)), ((), ())),
                        preferred_element_type=jnp.float32,
                    )
                    e = jnp.where(mask, jnp.exp(s), 0.0)
                    w = (e * (1.0 / jnp.sum(e, axis=1, keepdims=True))
                         ).astype(jnp.bfloat16)
                    ctx.append(lax.dot_general(
                        w, v_grp[:, sl], (((1,), (0,)), ((), ())),
                        preferred_element_type=jnp.float32,
                    ).astype(jnp.bfloat16))
            ctx2 = jnp.concatenate(
                [jnp.concatenate(ctx[:H_GRP], axis=1),
                 jnp.concatenate(ctx[H_GRP:], axis=1)], axis=0)
            return lax.dot_general(
                ctx2, wo_chunk, (((1,), (0,)), ((), ())),
                preferred_element_type=jnp.float32,
            )

        acc = group_contrib(my_pos, comm_wq[0], comm_wo[0])

        sends[0].wait_recv()
        sends[1].wait_recv()
        acc = acc + group_contrib(left, comm_wq[1], comm_wo[1])

        sends[2].wait_recv()
        sends[3].wait_recv()
        acc = acc + group_contrib(right, comm_wq[2], comm_wo[2])

        sends[4].wait_recv()
        sends[5].wait_recv()
        acc = acc + group_contrib(across, comm_wq[3], comm_wo[3])

        out_ref[0] = acc[:SQ]
        out_ref[1] = acc[SQ:]

        for d in sends:
            d.wait_send()

    return pl.pallas_call(
        body,
        out_shape=jax.ShapeDtypeStruct((B_LOC, SQ, D_MODEL), jnp.float32),
        in_specs=[pl.BlockSpec(memory_space=pltpu.VMEM)] * 5,
        out_specs=pl.BlockSpec(memory_space=pltpu.VMEM),
        scratch_shapes=[
            pltpu.VMEM((N_DEV, D_MODEL, GRP_W), jnp.bfloat16),
            pltpu.VMEM((N_DEV, GRP_W, D_MODEL), jnp.bfloat16),
            pltpu.SemaphoreType.DMA((3,)),
            pltpu.SemaphoreType.DMA((3,)),
            pltpu.SemaphoreType.DMA((3,)),
            pltpu.SemaphoreType.DMA((3,)),
        ],
        compiler_params=pltpu.CompilerParams(collective_id=0),
    )(x, Wq, K2, V2, Wo)
